# Initial kernel scaffold; baseline (speedup 1.0000x reference)
import jax
import jax.numpy as jnp
from jax.experimental import pallas as pl


def kernel(x, edge_index, W1, b1, W2, b2, Wd1, bd1, Wout, bout):
    raise NotImplementedError("write your pallas kernel here")


# speedup vs baseline: 15.1838x; 15.1838x over previous
import functools

import jax
import jax.numpy as jnp
from jax import lax
from jax.experimental import pallas as pl
from jax.experimental.pallas import tpu as pltpu
from jax.experimental.pallas import tpu_sc as plsc

_NC = 2
_NS = 16
_NW = _NC * _NS
_DEG_W = 16


def _chunk_size(edges_per_worker: int) -> int:
    for ch in range(128, 7, -8):
        if edges_per_worker % ch == 0:
            return ch
    raise ValueError(f"no valid chunk size for {edges_per_worker}")


def _sc_degree(dst, ones_rows, zeros_slab, n_pad):
    n_edges = dst.shape[0]
    epw = n_edges // _NW
    ch = _chunk_size(epw)
    nchunk = epw // ch
    rpt = n_pad // _NS
    mesh = plsc.VectorSubcoreMesh(core_axis_name="c", subcore_axis_name="s")

    @functools.partial(
        pl.kernel,
        out_type=jax.ShapeDtypeStruct((_NC, n_pad, _DEG_W), jnp.float32),
        mesh=mesh,
        scratch_types=[
            pltpu.VMEM((ch,), jnp.int32),
            pltpu.VMEM((ch, _DEG_W), jnp.float32),
            pltpu.VMEM((rpt, _DEG_W), jnp.float32),
            pltpu.VMEM_SHARED((n_pad, _DEG_W), jnp.float32),
        ],
        compiler_params=pltpu.CompilerParams(use_tc_tiling_on_sc=False),
    )
    def k(dst_hbm, ones_hbm, zeros_hbm, out_hbm, dstv, onesv, slab, acc):
        c = lax.axis_index("c")
        sid = lax.axis_index("s")
        wid = sid * _NC + c
        rbase = sid * rpt
        pltpu.sync_copy(zeros_hbm, slab)
        pltpu.sync_copy(slab, acc.at[pl.ds(rbase, rpt)])
        pltpu.sync_copy(ones_hbm, onesv)
        plsc.subcore_barrier()
        ebase = wid * epw

        def step(i, carry):
            pltpu.sync_copy(dst_hbm.at[pl.ds(ebase + i * ch, ch)], dstv)
            pltpu.sync_copy(onesv, acc.at[dstv], add=True)
            return carry

        lax.fori_loop(0, nchunk, step, 0)
        plsc.subcore_barrier()
        pltpu.sync_copy(acc.at[pl.ds(rbase, rpt)], slab)
        pltpu.sync_copy(slab, out_hbm.at[c, pl.ds(rbase, rpt)])

    return k(dst, ones_rows, zeros_slab)


def _sc_propagate(src, dst, g, zeros_slab, n_pad):
    d = g.shape[1]
    n_edges = src.shape[0]
    epw = n_edges // _NW
    ch = _chunk_size(epw)
    nchunk = epw // ch
    rpt = n_pad // _NS
    mesh = plsc.VectorSubcoreMesh(core_axis_name="c", subcore_axis_name="s")

    @functools.partial(
        pl.kernel,
        out_type=jax.ShapeDtypeStruct((_NC, n_pad, d), jnp.float32),
        mesh=mesh,
        scratch_types=[
            pltpu.VMEM((ch,), jnp.int32),
            pltpu.VMEM((ch,), jnp.int32),
            pltpu.VMEM((ch, d), jnp.float32),
            pltpu.VMEM((rpt, d), jnp.float32),
            pltpu.VMEM_SHARED((n_pad, d), jnp.float32),
            pltpu.SemaphoreType.DMA,
        ],
        compiler_params=pltpu.CompilerParams(use_tc_tiling_on_sc=False),
    )
    def k(src_hbm, dst_hbm, g_hbm, zeros_hbm, out_hbm,
          srcv, dstv, rows, slab, acc, sem):
        c = lax.axis_index("c")
        sid = lax.axis_index("s")
        wid = sid * _NC + c
        rbase = sid * rpt
        pltpu.sync_copy(zeros_hbm, slab)
        pltpu.sync_copy(slab, acc.at[pl.ds(rbase, rpt)])
        plsc.subcore_barrier()
        ebase = wid * epw

        def step(i, carry):
            off = ebase + i * ch
            pltpu.sync_copy(src_hbm.at[pl.ds(off, ch)], srcv)
            pltpu.sync_copy(dst_hbm.at[pl.ds(off, ch)], dstv)
            pltpu.async_copy(g_hbm.at[srcv], rows, sem).wait()
            pltpu.sync_copy(rows, acc.at[dstv], add=True)
            return carry

        lax.fori_loop(0, nchunk, step, 0)
        plsc.subcore_barrier()
        pltpu.sync_copy(acc.at[pl.ds(rbase, rpt)], slab)
        pltpu.sync_copy(slab, out_hbm.at[c, pl.ds(rbase, rpt)])

    return k(src, dst, g, zeros_slab)


def _tc_prep(x, W1, degp):
    n = x.shape[0]

    def body(x_ref, w_ref, degp_ref, g_ref, s_ref):
        deg = degp_ref[0, :x_ref.shape[0]] + degp_ref[1, :x_ref.shape[0]]
        s = lax.rsqrt(deg[:, 0:1] + 1.0)
        s_ref[...] = s
        g_ref[...] = jnp.dot(x_ref[...], w_ref[...],
                             preferred_element_type=jnp.float32) * s

    return pl.pallas_call(
        body,
        out_shape=(
            jax.ShapeDtypeStruct((n, W1.shape[1]), jnp.float32),
            jax.ShapeDtypeStruct((n, 1), jnp.float32),
        ),
    )(x, W1, degp)


def _tc_mid(acc, g, s, b, W2):
    n = g.shape[0]

    def body(acc_ref, g_ref, s_ref, b_ref, w_ref, out_ref):
        s_ = s_ref[...]
        z = jnp.maximum(
            s_ * (acc_ref[0, :n] + acc_ref[1, :n] + g_ref[...]) + b_ref[...],
            0.0)
        out_ref[...] = jnp.dot(z, w_ref[...],
                               preferred_element_type=jnp.float32) * s_

    return pl.pallas_call(
        body,
        out_shape=jax.ShapeDtypeStruct((n, W2.shape[1]), jnp.float32),
    )(acc, g, s, b, W2)


def _tc_final(acc, g, s, b2, Wd1, bd1, Wout, bout):
    n = g.shape[0]

    def body(acc_ref, g_ref, s_ref, b2_ref, wd1_ref, bd1_ref, wout_ref,
             bout_ref, out_ref):
        z2 = jnp.maximum(
            s_ref[...] * (acc_ref[0, :n] + acc_ref[1, :n] + g_ref[...])
            + b2_ref[...],
            0.0)
        h = jnp.maximum(
            jnp.dot(z2, wd1_ref[...], preferred_element_type=jnp.float32)
            + bd1_ref[...], 0.0)
        out_ref[...] = (
            jnp.dot(h, wout_ref[...], preferred_element_type=jnp.float32)
            + bout_ref[...])

    return pl.pallas_call(
        body,
        out_shape=jax.ShapeDtypeStruct((n, Wout.shape[1]), jnp.float32),
    )(acc, g, s, b2, Wd1, bd1, Wout, bout)


def kernel(x, edge_index, W1, b1, W2, b2, Wd1, bd1, Wout, bout):
    n = x.shape[0]
    src = edge_index[0].astype(jnp.int32)
    dst = edge_index[1].astype(jnp.int32)
    n_edges = src.shape[0]
    ch = _chunk_size(n_edges // _NW)
    n_pad = -(-n // (_NS * 8)) * (_NS * 8)
    rpt = n_pad // _NS

    ones_rows = jnp.ones((ch, _DEG_W), jnp.float32)
    degp = _sc_degree(dst, ones_rows, jnp.zeros((rpt, _DEG_W), jnp.float32),
                      n_pad)
    g1, s = _tc_prep(x, W1, degp)
    acc1 = _sc_propagate(src, dst, g1, jnp.zeros((rpt, 64), jnp.float32),
                         n_pad)
    g2 = _tc_mid(acc1, g1, s, b1.reshape(1, -1), W2)
    acc2 = _sc_propagate(src, dst, g2, jnp.zeros((rpt, 32), jnp.float32),
                         n_pad)
    return _tc_final(acc2, g2, s, b2.reshape(1, -1), Wd1,
                     bd1.reshape(1, -1), Wout, bout.reshape(1, -1))


# speedup vs baseline: 31.8835x; 2.0998x over previous
import functools

import jax
import jax.numpy as jnp
from jax import lax
from jax.experimental import pallas as pl
from jax.experimental.pallas import tpu as pltpu
from jax.experimental.pallas import tpu_sc as plsc

_NC = 2
_NS = 16
_NW = _NC * _NS
_DEG_W = 16


def _chunk_size(edges_per_worker: int) -> int:
    for ch in range(128, 3, -4):
        if edges_per_worker % ch == 0 and (edges_per_worker // ch) % 2 == 0:
            return ch
    raise ValueError(f"no valid chunk size for {edges_per_worker}")


def _sc_degree(dst2d, ones_rows, zeros_slab, n_pad):
    nrows, ch = dst2d.shape
    nchunk = nrows // _NW
    rpt = n_pad // _NS
    mesh = plsc.VectorSubcoreMesh(core_axis_name="c", subcore_axis_name="s")

    @functools.partial(
        pl.kernel,
        out_type=jax.ShapeDtypeStruct((_NC, n_pad, _DEG_W), jnp.float32),
        mesh=mesh,
        scratch_types=[
            pltpu.VMEM((nchunk, ch), jnp.int32),
            pltpu.VMEM((ch, _DEG_W), jnp.float32),
            pltpu.VMEM((rpt, _DEG_W), jnp.float32),
            pltpu.VMEM_SHARED((n_pad, _DEG_W), jnp.float32),
            pltpu.SemaphoreType.DMA,
        ],
        compiler_params=pltpu.CompilerParams(use_tc_tiling_on_sc=False),
    )
    def k(dst_hbm, ones_hbm, zeros_hbm, out_hbm, dstv, onesv, slab, acc, sem):
        c = lax.axis_index("c")
        sid = lax.axis_index("s")
        wid = sid * _NC + c
        rbase = sid * rpt
        pltpu.sync_copy(zeros_hbm, slab)
        pltpu.sync_copy(slab, acc.at[pl.ds(rbase, rpt)])
        pltpu.sync_copy(ones_hbm, onesv)
        pltpu.sync_copy(dst_hbm.at[pl.ds(wid * nchunk, nchunk)], dstv)
        plsc.subcore_barrier()

        pltpu.async_copy(onesv, acc.at[dstv.at[0]], sem, add=True)

        def step(i, carry):
            pltpu.async_copy(onesv, acc.at[dstv.at[i + 1]], sem, add=True)
            pltpu.make_async_copy(onesv, acc.at[dstv.at[i]], sem).wait()
            return carry

        lax.fori_loop(0, nchunk - 1, step, 0)
        pltpu.make_async_copy(onesv, acc.at[dstv.at[0]], sem).wait()
        plsc.subcore_barrier()
        pltpu.sync_copy(acc.at[pl.ds(rbase, rpt)], slab)
        pltpu.sync_copy(slab, out_hbm.at[c, pl.ds(rbase, rpt)])

    return k(dst2d, ones_rows, zeros_slab)


def _sc_propagate(src2d, dst2d, g, zeros_slab, n_pad):
    d = g.shape[1]
    nrows, ch = src2d.shape
    nchunk = nrows // _NW
    rpt = n_pad // _NS
    mesh = plsc.VectorSubcoreMesh(core_axis_name="c", subcore_axis_name="s")

    @functools.partial(
        pl.kernel,
        out_type=jax.ShapeDtypeStruct((_NC, n_pad, d), jnp.float32),
        mesh=mesh,
        scratch_types=[
            pltpu.VMEM((nchunk, ch), jnp.int32),
            pltpu.VMEM((nchunk, ch), jnp.int32),
            pltpu.VMEM((ch, d), jnp.float32),
            pltpu.VMEM((ch, d), jnp.float32),
            pltpu.VMEM((rpt, d), jnp.float32),
            pltpu.VMEM_SHARED((n_pad, d), jnp.float32),
            pltpu.SemaphoreType.DMA,
            pltpu.SemaphoreType.DMA,
            pltpu.SemaphoreType.DMA,
            pltpu.SemaphoreType.DMA,
        ],
        compiler_params=pltpu.CompilerParams(use_tc_tiling_on_sc=False),
    )
    def k(src_hbm, dst_hbm, g_hbm, zeros_hbm, out_hbm,
          srcv, dstv, rows0, rows1, slab, acc, gsem0, gsem1, ssem0, ssem1):
        c = lax.axis_index("c")
        sid = lax.axis_index("s")
        wid = sid * _NC + c
        rbase = sid * rpt
        pltpu.sync_copy(zeros_hbm, slab)
        pltpu.sync_copy(slab, acc.at[pl.ds(rbase, rpt)])
        pltpu.sync_copy(src_hbm.at[pl.ds(wid * nchunk, nchunk)], srcv)
        pltpu.sync_copy(dst_hbm.at[pl.ds(wid * nchunk, nchunk)], dstv)
        plsc.subcore_barrier()

        rows = (rows0, rows1)
        gsem = (gsem0, gsem1)
        ssem = (ssem0, ssem1)

        pltpu.async_copy(g_hbm.at[srcv.at[0]], rows0, gsem0)

        def pair(j, carry):
            for b in (0, 1):
                i = 2 * j + b
                nb = 1 - b
                pltpu.make_async_copy(g_hbm.at[srcv.at[i]], rows[b],
                                      gsem[b]).wait()
                if b == 0:
                    @pl.when(j > 0)
                    def _():
                        pltpu.make_async_copy(rows[nb], acc.at[dstv.at[i]],
                                              ssem[nb]).wait()
                    pltpu.async_copy(g_hbm.at[srcv.at[i + 1]], rows[nb],
                                     gsem[nb])
                else:
                    pltpu.make_async_copy(rows[nb], acc.at[dstv.at[i]],
                                          ssem[nb]).wait()

                    @pl.when(j < nchunk // 2 - 1)
                    def _():
                        pltpu.async_copy(g_hbm.at[srcv.at[i + 1]], rows[nb],
                                         gsem[nb])
                pltpu.async_copy(rows[b], acc.at[dstv.at[i]], ssem[b],
                                 add=True)
            return carry

        lax.fori_loop(0, nchunk // 2, pair, 0)
        pltpu.make_async_copy(rows1, acc.at[dstv.at[0]], ssem1).wait()
        plsc.subcore_barrier()
        pltpu.sync_copy(acc.at[pl.ds(rbase, rpt)], slab)
        pltpu.sync_copy(slab, out_hbm.at[c, pl.ds(rbase, rpt)])

    return k(src2d, dst2d, g, zeros_slab)


def _tc_prep(x, W1, degp):
    n = x.shape[0]

    def body(x_ref, w_ref, degp_ref, g_ref, s_ref):
        deg = degp_ref[0, :x_ref.shape[0]] + degp_ref[1, :x_ref.shape[0]]
        s = lax.rsqrt(deg[:, 0:1] + 1.0)
        s_ref[...] = s
        g_ref[...] = jnp.dot(x_ref[...], w_ref[...],
                             preferred_element_type=jnp.float32) * s

    return pl.pallas_call(
        body,
        out_shape=(
            jax.ShapeDtypeStruct((n, W1.shape[1]), jnp.float32),
            jax.ShapeDtypeStruct((n, 1), jnp.float32),
        ),
    )(x, W1, degp)


def _tc_mid(acc, g, s, b, W2):
    n = g.shape[0]

    def body(acc_ref, g_ref, s_ref, b_ref, w_ref, out_ref):
        s_ = s_ref[...]
        z = jnp.maximum(
            s_ * (acc_ref[0, :n] + acc_ref[1, :n] + g_ref[...]) + b_ref[...],
            0.0)
        out_ref[...] = jnp.dot(z, w_ref[...],
                               preferred_element_type=jnp.float32) * s_

    return pl.pallas_call(
        body,
        out_shape=jax.ShapeDtypeStruct((n, W2.shape[1]), jnp.float32),
    )(acc, g, s, b, W2)


def _tc_final(acc, g, s, b2, Wd1, bd1, Wout, bout):
    n = g.shape[0]

    def body(acc_ref, g_ref, s_ref, b2_ref, wd1_ref, bd1_ref, wout_ref,
             bout_ref, out_ref):
        z2 = jnp.maximum(
            s_ref[...] * (acc_ref[0, :n] + acc_ref[1, :n] + g_ref[...])
            + b2_ref[...],
            0.0)
        h = jnp.maximum(
            jnp.dot(z2, wd1_ref[...], preferred_element_type=jnp.float32)
            + bd1_ref[...], 0.0)
        out_ref[...] = (
            jnp.dot(h, wout_ref[...], preferred_element_type=jnp.float32)
            + bout_ref[...])

    return pl.pallas_call(
        body,
        out_shape=jax.ShapeDtypeStruct((n, Wout.shape[1]), jnp.float32),
    )(acc, g, s, b2, Wd1, bd1, Wout, bout)


def kernel(x, edge_index, W1, b1, W2, b2, Wd1, bd1, Wout, bout):
    n = x.shape[0]
    n_edges = edge_index.shape[1]
    ch = _chunk_size(n_edges // _NW)
    src2d = edge_index[0].astype(jnp.int32).reshape(n_edges // ch, ch)
    dst2d = edge_index[1].astype(jnp.int32).reshape(n_edges // ch, ch)
    n_pad = -(-n // (_NS * 8)) * (_NS * 8)
    rpt = n_pad // _NS

    ones_rows = jnp.ones((ch, _DEG_W), jnp.float32)
    degp = _sc_degree(dst2d, ones_rows, jnp.zeros((rpt, _DEG_W), jnp.float32),
                      n_pad)
    g1, s = _tc_prep(x, W1, degp)
    acc1 = _sc_propagate(src2d, dst2d, g1, jnp.zeros((rpt, 64), jnp.float32),
                         n_pad)
    g2 = _tc_mid(acc1, g1, s, b1.reshape(1, -1), W2)
    acc2 = _sc_propagate(src2d, dst2d, g2, jnp.zeros((rpt, 32), jnp.float32),
                         n_pad)
    return _tc_final(acc2, g2, s, b2.reshape(1, -1), Wd1,
                     bd1.reshape(1, -1), Wout, bout.reshape(1, -1))
